# baseline (speedup 1.0000x reference)
import jax
import jax.numpy as jnp
from jax.experimental import pallas as pl


def _dg0(w, x):
    return jax.lax.dot_general(w, x, (((0,), (0,)), ((), ())),
                               preferred_element_type=jnp.float32)


def _body(vel_ref, pos_ref, hed_ref, speed_ref, ef_ref, pk_ref, bcol_ref,
          h_ref, p_ref, d_ref, e_ref):
    vel = vel_ref[...]
    pos = pos_ref[...]
    hed = hed_ref[...]
    sp = speed_ref[...]
    ones = jnp.ones(sp.shape, jnp.float32)
    nf8 = jnp.concatenate([vel, pos, hed, sp, ones], axis=0)
    pk = pk_ref[...]

    h_ref[...] = _dg0(pk[:, 0:64], nf8)
    p_ref[...] = _dg0(pk[:, 64:128], nf8)

    rs3 = jnp.concatenate([jnp.round(hed) * sp, ones], axis=0)
    d_ref[...] = _dg0(pk[0:3, 128:192], rs3)

    e_ref[...] = _dg0(pk[0:4, 192:256], ef_ref[...]) + bcol_ref[...][:, 0:1]


def kernel(vel, pos, hed, speed, e_feat, edge_index,
           W_h, b_h, W_p, b_p, W_d, b_d, W_e, b_e):
    del edge_index
    n = vel.shape[0]
    e_rows = e_feat.shape[0]
    hdim = W_h.shape[1]
    f32 = jnp.float32

    z1 = jnp.zeros((1, hdim), f32)
    z2 = jnp.zeros((2, hdim), f32)
    c_h = jnp.concatenate([W_h, b_h[None, :]], axis=0)
    c_p = jnp.concatenate([z2, W_p, z2, z1, b_p[None, :]], axis=0)
    c_d = jnp.concatenate([W_d, b_d[None, :], z2, z2, z1], axis=0)
    c_e = jnp.concatenate([W_e, z2, z2], axis=0)
    pk = jnp.concatenate([c_h, c_p, c_d, c_e], axis=1)
    bcol = b_e.reshape(hdim, 1)

    grid = 25
    bn = 2048
    be = e_rows // grid

    def cmap(i):
        return (0, i)

    def wmap(i):
        return (0, 0)

    out = pl.pallas_call(
        _body,
        grid=(grid,),
        in_specs=[
            pl.BlockSpec((2, bn), cmap),
            pl.BlockSpec((2, bn), cmap),
            pl.BlockSpec((2, bn), cmap),
            pl.BlockSpec((1, bn), cmap),
            pl.BlockSpec((4, be), cmap),
            pl.BlockSpec((8, 4 * hdim), wmap),
            pl.BlockSpec((hdim, 1), wmap),
        ],
        out_specs=[
            pl.BlockSpec((hdim, bn), cmap),
            pl.BlockSpec((hdim, bn), cmap),
            pl.BlockSpec((hdim, bn), cmap),
            pl.BlockSpec((hdim, be), cmap),
        ],
        out_shape=[
            jax.ShapeDtypeStruct((hdim, n), f32),
            jax.ShapeDtypeStruct((hdim, n), f32),
            jax.ShapeDtypeStruct((hdim, n), f32),
            jax.ShapeDtypeStruct((hdim, e_rows), f32),
        ],
    )(vel.T, pos.T, hed.T, speed.T, e_feat.T, pk, bcol)

    return (out[0].T, out[1].T, out[2].T, out[3].T)


# speedup vs baseline: 1.0177x; 1.0177x over previous
import jax
import jax.numpy as jnp
from jax.experimental import pallas as pl


def _dg0(w, x):
    return jax.lax.dot_general(w, x, (((0,), (0,)), ((), ())),
                               preferred_element_type=jnp.float32)


def _body(vel_ref, pos_ref, hed_ref, speed_ref, ef_ref, pk_ref,
          h_ref, p_ref, d_ref, e_ref):
    vel = vel_ref[...]
    pos = pos_ref[...]
    hed = hed_ref[...]
    sp = speed_ref[...]
    ones = jnp.ones(sp.shape, jnp.float32)
    nf8 = jnp.concatenate([vel, pos, hed, sp, ones], axis=0)
    pk = pk_ref[...]

    h_ref[...] = _dg0(pk[:, 0:64], nf8)
    p_ref[...] = _dg0(pk[:, 64:128], nf8)

    rs3 = jnp.concatenate([jnp.round(hed) * sp, ones], axis=0)
    d_ref[...] = _dg0(pk[0:3, 128:192], rs3)

    ef = ef_ref[...]
    efc = jnp.concatenate(
        [ef, jnp.ones((1, ef.shape[1]), jnp.float32)], axis=0)
    e_ref[...] = _dg0(pk[0:5, 192:256], efc)


def kernel(vel, pos, hed, speed, e_feat, edge_index,
           W_h, b_h, W_p, b_p, W_d, b_d, W_e, b_e):
    del edge_index
    n = vel.shape[0]
    e_rows = e_feat.shape[0]
    hdim = W_h.shape[1]
    f32 = jnp.float32

    z1 = jnp.zeros((1, hdim), f32)
    z2 = jnp.zeros((2, hdim), f32)
    c_h = jnp.concatenate([W_h, b_h[None, :]], axis=0)
    c_p = jnp.concatenate([z2, W_p, z2, z1, b_p[None, :]], axis=0)
    c_d = jnp.concatenate([W_d, b_d[None, :], z2, z2, z1], axis=0)
    c_e = jnp.concatenate([W_e, b_e[None, :], z1, z2], axis=0)
    pk = jnp.concatenate([c_h, c_p, c_d, c_e], axis=1)

    grid = 25
    bn = 2048
    be = e_rows // grid

    def cmap(i):
        return (0, i)

    def wmap(i):
        return (0, 0)

    out = pl.pallas_call(
        _body,
        grid=(grid,),
        in_specs=[
            pl.BlockSpec((2, bn), cmap),
            pl.BlockSpec((2, bn), cmap),
            pl.BlockSpec((2, bn), cmap),
            pl.BlockSpec((1, bn), cmap),
            pl.BlockSpec((4, be), cmap),
            pl.BlockSpec((8, 4 * hdim), wmap),
        ],
        out_specs=[
            pl.BlockSpec((hdim, bn), cmap),
            pl.BlockSpec((hdim, bn), cmap),
            pl.BlockSpec((hdim, bn), cmap),
            pl.BlockSpec((hdim, be), cmap),
        ],
        out_shape=[
            jax.ShapeDtypeStruct((hdim, n), f32),
            jax.ShapeDtypeStruct((hdim, n), f32),
            jax.ShapeDtypeStruct((hdim, n), f32),
            jax.ShapeDtypeStruct((hdim, e_rows), f32),
        ],
    )(vel.T, pos.T, hed.T, speed.T, e_feat.T, pk)

    return (out[0].T, out[1].T, out[2].T, out[3].T)


# speedup vs baseline: 1.0503x; 1.0321x over previous
import jax
import jax.numpy as jnp
from jax.experimental import pallas as pl


def _dg0(w, x):
    return jax.lax.dot_general(w, x, (((0,), (0,)), ((), ())),
                               preferred_element_type=jnp.float32)


def _body(vel_ref, pos_ref, hed_ref, speed_ref, ef_ref,
          wh_ref, bh_ref, wp_ref, bp_ref, wd_ref, bd_ref, we_ref, be_ref,
          h_ref, p_ref, d_ref, e_ref):
    vel = vel_ref[...]
    pos = pos_ref[...]
    hed = hed_ref[...]
    sp = speed_ref[...]
    ones = jnp.ones(sp.shape, jnp.float32)
    nf8 = jnp.concatenate([vel, pos, hed, sp, ones], axis=0)

    w_h8 = jnp.concatenate([wh_ref[...], bh_ref[...]], axis=0)
    h_ref[...] = _dg0(w_h8, nf8)

    w_p3 = jnp.concatenate([wp_ref[...], bp_ref[...]], axis=0)
    p_ref[...] = _dg0(w_p3, jnp.concatenate([pos, ones], axis=0))

    rs3 = jnp.concatenate([jnp.round(hed) * sp, ones], axis=0)
    w_d3 = jnp.concatenate([wd_ref[...], bd_ref[...]], axis=0)
    d_ref[...] = _dg0(w_d3, rs3)

    ef = ef_ref[...]
    efc = jnp.concatenate(
        [ef, jnp.ones((1, ef.shape[1]), jnp.float32)], axis=0)
    w_e5 = jnp.concatenate([we_ref[...], be_ref[...]], axis=0)
    e_ref[...] = _dg0(w_e5, efc)


def kernel(vel, pos, hed, speed, e_feat, edge_index,
           W_h, b_h, W_p, b_p, W_d, b_d, W_e, b_e):
    del edge_index
    n = vel.shape[0]
    e_rows = e_feat.shape[0]
    hdim = W_h.shape[1]
    f32 = jnp.float32


    grid = 25
    bn = 2048
    be = e_rows // grid

    def cmap(i):
        return (0, i)

    def wmap(i):
        return (0, 0)

    out = pl.pallas_call(
        _body,
        grid=(grid,),
        in_specs=[
            pl.BlockSpec((2, bn), cmap),
            pl.BlockSpec((2, bn), cmap),
            pl.BlockSpec((2, bn), cmap),
            pl.BlockSpec((1, bn), cmap),
            pl.BlockSpec((4, be), cmap),
            pl.BlockSpec((7, hdim), wmap),
            pl.BlockSpec((1, hdim), wmap),
            pl.BlockSpec((2, hdim), wmap),
            pl.BlockSpec((1, hdim), wmap),
            pl.BlockSpec((2, hdim), wmap),
            pl.BlockSpec((1, hdim), wmap),
            pl.BlockSpec((4, hdim), wmap),
            pl.BlockSpec((1, hdim), wmap),
        ],
        out_specs=[
            pl.BlockSpec((hdim, bn), cmap),
            pl.BlockSpec((hdim, bn), cmap),
            pl.BlockSpec((hdim, bn), cmap),
            pl.BlockSpec((hdim, be), cmap),
        ],
        out_shape=[
            jax.ShapeDtypeStruct((hdim, n), f32),
            jax.ShapeDtypeStruct((hdim, n), f32),
            jax.ShapeDtypeStruct((hdim, n), f32),
            jax.ShapeDtypeStruct((hdim, e_rows), f32),
        ],
    )(vel.T, pos.T, hed.T, speed.T, e_feat.T,
      W_h, b_h[None, :], W_p, b_p[None, :], W_d, b_d[None, :],
      W_e, b_e[None, :])

    return (out[0].T, out[1].T, out[2].T, out[3].T)
